# baseline (speedup 1.0000x reference)
import functools

import jax
import jax.numpy as jnp
from jax import lax
from jax.experimental import pallas as pl
from jax.experimental.pallas import tpu as pltpu
from jax.experimental.pallas import tpu_sc as plsc

N = 4096
D = 64
K1 = 11
PAD = 16
E = N * K1
R = 1024
G = N // R
PADIDX = 2 ** 30

NC = 2
NS = 16
NW = NC * NS
ROWS_PER_W = N // NW
CHUNK = ROWS_PER_W * PAD
ECHUNK = ROWS_PER_W * K1
NGATHER = ECHUNK // PAD


def _normalize_body(mv_ref, normed_ref):
    mv = mv_ref[:]
    nf = jnp.sqrt(jnp.sum(mv * mv, axis=1, keepdims=True))
    normed_ref[:] = mv / jnp.clip(nf, 1e-6, None)


_normalize = pl.pallas_call(
    _normalize_body,
    out_shape=jax.ShapeDtypeStruct((N, D), jnp.float32),
)


def _topk_body(normed_full_ref, normed_rows_ref, val_ref, idx_ref):
    normed_full = normed_full_ref[:]
    normed_rows = normed_rows_ref[:]
    sim = lax.dot_general(normed_rows, normed_full,
                          (((1,), (1,)), ((), ())),
                          preferred_element_type=jnp.float32)
    colf = lax.broadcasted_iota(jnp.int32, (R, N), 1).astype(jnp.float32)
    i = pl.program_id(0)
    rowv = (lax.broadcasted_iota(jnp.int32, (R, 1), 0) + i * R)
    rowvf = rowv.astype(jnp.float32)
    diag = jnp.sum(normed_rows * normed_rows, axis=1, keepdims=True)
    vals, idxs = [diag], [rowvf]
    sim = jnp.where(colf == rowvf, -jnp.inf, sim)
    for _ in range(K1 - 1):
        m = jnp.max(sim, axis=1, keepdims=True)
        aminf = jnp.min(jnp.where(sim == m, colf, jnp.float32(2 * N)),
                        axis=1, keepdims=True)
        vals.append(m)
        idxs.append(aminf)
        sim = jnp.where(colf == aminf, -jnp.inf, sim)
    vals += [jnp.zeros((R, 1), jnp.float32)] * (PAD - K1)
    idxs += [jnp.full((R, 1), float(PADIDX), jnp.float32)] * (PAD - K1)
    val_ref[:] = jnp.concatenate(vals, axis=1)
    idx_ref[:] = jnp.concatenate(idxs, axis=1).astype(jnp.int32)


_topk = pl.pallas_call(
    _topk_body,
    grid=(G,),
    in_specs=[pl.BlockSpec((N, D), lambda i: (0, 0)),
              pl.BlockSpec((R, D), lambda i: (i, 0))],
    out_specs=[pl.BlockSpec((R, PAD), lambda i: (i, 0)),
               pl.BlockSpec((R, PAD), lambda i: (i, 0))],
    out_shape=[jax.ShapeDtypeStruct((N, PAD), jnp.float32),
               jax.ShapeDtypeStruct((N, PAD), jnp.int32)],
)


@functools.lru_cache(maxsize=1)
def _make_sc_finish():
    @functools.partial(
        pl.kernel,
        out_type=[jax.ShapeDtypeStruct((2 * E,), jnp.int32),
                  jax.ShapeDtypeStruct((E,), jnp.float32)],
        mesh=plsc.VectorSubcoreMesh(core_axis_name="c", subcore_axis_name="s"),
        compiler_params=pltpu.CompilerParams(needs_layout_passes=False),
        scratch_types=[pltpu.VMEM((CHUNK,), jnp.int32),
                       pltpu.VMEM((CHUNK,), jnp.float32),
                       pltpu.VMEM((CHUNK,), jnp.int32),
                       pltpu.VMEM((CHUNK,), jnp.float32),
                       pltpu.VMEM((ECHUNK,), jnp.int32),
                       pltpu.VMEM((ECHUNK,), jnp.int32),
                       pltpu.VMEM((ECHUNK,), jnp.int32),
                       pltpu.VMEM((ECHUNK,), jnp.float32)],
    )
    def _sc_finish(idx_hbm, val_hbm, gmap_hbm, rmap_hbm, ei_hbm, ew_hbm,
                   idx_v, val_v, sidx_v, sval_v,
                   gmap_v, rows_v, cols_v, w_v):
        wid = lax.axis_index("s") * NC + lax.axis_index("c")
        base = wid * CHUNK
        ebase = wid * ECHUNK
        pltpu.sync_copy(idx_hbm.at[pl.ds(base, CHUNK)], idx_v)
        pltpu.sync_copy(val_hbm.at[pl.ds(base, CHUNK)], val_v)
        pltpu.sync_copy(gmap_hbm, gmap_v)
        pltpu.sync_copy(rmap_hbm, rows_v)

        def sort_body(r, carry):
            off = pl.multiple_of(r * PAD, PAD)
            k = idx_v[pl.ds(off, PAD)]
            v = val_v[pl.ds(off, PAD)]
            ks, vs = plsc.sort_key_val(k, v)
            sidx_v[pl.ds(off, PAD)] = ks
            sval_v[pl.ds(off, PAD)] = vs
            return carry

        lax.fori_loop(0, ROWS_PER_W, sort_body, 0)

        row0 = (wid * ROWS_PER_W).astype(jnp.int32)

        def gather_body(j, carry):
            off = pl.multiple_of(j * PAD, PAD)
            g = gmap_v[pl.ds(off, PAD)]
            cols_v[pl.ds(off, PAD)] = plsc.load_gather(sidx_v, [g])
            w_v[pl.ds(off, PAD)] = plsc.load_gather(sval_v, [g])
            rows_v[pl.ds(off, PAD)] = rows_v[pl.ds(off, PAD)] + row0
            return carry

        lax.fori_loop(0, NGATHER, gather_body, 0)

        pltpu.sync_copy(rows_v, ei_hbm.at[pl.ds(ebase, ECHUNK)])
        pltpu.sync_copy(cols_v, ei_hbm.at[pl.ds(E + ebase, ECHUNK)])
        pltpu.sync_copy(w_v, ew_hbm.at[pl.ds(ebase, ECHUNK)])

    return _sc_finish


def kernel(memory_value):
    normed = _normalize(memory_value)
    vals, idxs = _topk(normed, normed)
    p = jnp.arange(ECHUNK, dtype=jnp.int32)
    gmap = p + (PAD - K1) * (p // K1)
    rmap = p // K1
    ei, ew = _make_sc_finish()(idxs.reshape(-1), vals.reshape(-1), gmap, rmap)
    return ei.reshape(2, E), ew


# speedup vs baseline: 1.3429x; 1.3429x over previous
import functools

import jax
import jax.numpy as jnp
from jax import lax
from jax.experimental import pallas as pl
from jax.experimental.pallas import tpu as pltpu
from jax.experimental.pallas import tpu_sc as plsc

N = 4096
D = 64
K1 = 11
PAD = 16
E = N * K1
R = 512
G = N // R
PADIDX = 2 ** 30

NC = 2
NS = 16
NW = NC * NS
ROWS_PER_W = N // NW
CHUNK = ROWS_PER_W * PAD
ECHUNK = ROWS_PER_W * K1
NGATHER = ECHUNK // PAD


def _topk_body(mv_ref, val_ref, idx_ref, normed_ref):
    i = pl.program_id(0)

    @pl.when(i == 0)
    def _():
        mv = mv_ref[:]
        nf = jnp.sqrt(jnp.sum(mv * mv, axis=1, keepdims=True))
        normed_ref[:] = mv / jnp.clip(nf, 1e-6, None)

    normed_full = normed_ref[:]
    normed_rows = normed_ref[pl.ds(i * R, R)]
    sim = lax.dot_general(normed_rows, normed_full,
                          (((1,), (1,)), ((), ())),
                          preferred_element_type=jnp.float32)
    colf = lax.broadcasted_iota(jnp.int32, (R, N), 1).astype(jnp.float32)
    rowv = (lax.broadcasted_iota(jnp.int32, (R, 1), 0) + i * R)
    rowvf = rowv.astype(jnp.float32)
    diag = jnp.sum(normed_rows * normed_rows, axis=1, keepdims=True)
    vals, idxs = [diag], [rowvf]
    sim = jnp.where(colf == rowvf, -jnp.inf, sim)
    for _ in range(K1 - 1):
        m = jnp.max(sim, axis=1, keepdims=True)
        aminf = jnp.min(jnp.where(sim == m, colf, jnp.float32(2 * N)),
                        axis=1, keepdims=True)
        vals.append(m)
        idxs.append(aminf)
        sim = jnp.where(colf == aminf, -jnp.inf, sim)
    vals += [jnp.zeros((R, 1), jnp.float32)] * (PAD - K1)
    idxs += [jnp.full((R, 1), float(PADIDX), jnp.float32)] * (PAD - K1)
    val_ref[:] = jnp.concatenate(vals, axis=1)
    idx_ref[:] = jnp.concatenate(idxs, axis=1).astype(jnp.int32)


_topk = pl.pallas_call(
    _topk_body,
    grid=(G,),
    in_specs=[pl.BlockSpec((N, D), lambda i: (0, 0))],
    out_specs=[pl.BlockSpec((R, PAD), lambda i: (i, 0)),
               pl.BlockSpec((R, PAD), lambda i: (i, 0))],
    out_shape=[jax.ShapeDtypeStruct((N, PAD), jnp.float32),
               jax.ShapeDtypeStruct((N, PAD), jnp.int32)],
    scratch_shapes=[pltpu.VMEM((N, D), jnp.float32)],
)


@functools.lru_cache(maxsize=1)
def _make_sc_finish():
    @functools.partial(
        pl.kernel,
        out_type=[jax.ShapeDtypeStruct((2 * E,), jnp.int32),
                  jax.ShapeDtypeStruct((E,), jnp.float32)],
        mesh=plsc.VectorSubcoreMesh(core_axis_name="c", subcore_axis_name="s"),
        compiler_params=pltpu.CompilerParams(needs_layout_passes=False),
        scratch_types=[pltpu.VMEM((CHUNK,), jnp.int32),
                       pltpu.VMEM((CHUNK,), jnp.float32),
                       pltpu.VMEM((CHUNK,), jnp.int32),
                       pltpu.VMEM((CHUNK,), jnp.float32),
                       pltpu.VMEM((ECHUNK,), jnp.int32),
                       pltpu.VMEM((ECHUNK,), jnp.int32),
                       pltpu.VMEM((ECHUNK,), jnp.int32),
                       pltpu.VMEM((ECHUNK,), jnp.float32)],
    )
    def _sc_finish(idx_hbm, val_hbm, gmap_hbm, rmap_hbm, ei_hbm, ew_hbm,
                   idx_v, val_v, sidx_v, sval_v,
                   gmap_v, rows_v, cols_v, w_v):
        wid = lax.axis_index("s") * NC + lax.axis_index("c")
        base = wid * CHUNK
        ebase = wid * ECHUNK
        pltpu.sync_copy(idx_hbm.at[pl.ds(base, CHUNK)], idx_v)
        pltpu.sync_copy(val_hbm.at[pl.ds(base, CHUNK)], val_v)
        pltpu.sync_copy(gmap_hbm, gmap_v)
        pltpu.sync_copy(rmap_hbm, rows_v)

        def sort_body(r, carry):
            off = pl.multiple_of(r * PAD, PAD)
            k = idx_v[pl.ds(off, PAD)]
            v = val_v[pl.ds(off, PAD)]
            ks, vs = plsc.sort_key_val(k, v)
            sidx_v[pl.ds(off, PAD)] = ks
            sval_v[pl.ds(off, PAD)] = vs
            return carry

        lax.fori_loop(0, ROWS_PER_W, sort_body, 0)

        row0 = (wid * ROWS_PER_W).astype(jnp.int32)

        def gather_body(j, carry):
            off = pl.multiple_of(j * PAD, PAD)
            g = gmap_v[pl.ds(off, PAD)]
            cols_v[pl.ds(off, PAD)] = plsc.load_gather(sidx_v, [g])
            w_v[pl.ds(off, PAD)] = plsc.load_gather(sval_v, [g])
            rows_v[pl.ds(off, PAD)] = rows_v[pl.ds(off, PAD)] + row0
            return carry

        lax.fori_loop(0, NGATHER, gather_body, 0)

        pltpu.sync_copy(rows_v, ei_hbm.at[pl.ds(ebase, ECHUNK)])
        pltpu.sync_copy(cols_v, ei_hbm.at[pl.ds(E + ebase, ECHUNK)])
        pltpu.sync_copy(w_v, ew_hbm.at[pl.ds(ebase, ECHUNK)])

    return _sc_finish


def kernel(memory_value):
    vals, idxs = _topk(memory_value)
    p = jnp.arange(ECHUNK, dtype=jnp.int32)
    gmap = p + (PAD - K1) * (p // K1)
    rmap = p // K1
    ei, ew = _make_sc_finish()(idxs.reshape(-1), vals.reshape(-1), gmap, rmap)
    return ei.reshape(2, E), ew
